# Initial kernel scaffold; baseline (speedup 1.0000x reference)
import jax
import jax.numpy as jnp
from jax.experimental import pallas as pl


def kernel(x, adj_indices, adj_values, edge_index, W1, W2, Wl, bl):
    raise NotImplementedError("write your pallas kernel here")


# speedup vs baseline: 5.5359x; 5.5359x over previous
import functools

import jax
import jax.numpy as jnp
from jax import lax
from jax.experimental import pallas as pl
from jax.experimental.pallas import tpu as pltpu
from jax.experimental.pallas import tpu_sc as plsc

N = 10000
NPAD = 10240
DIN = 128
DH = 64
L = 16
NC = 2
NS = 16
NW = NC * NS
CHUNK = 128
ROWS_PER_TILE = NPAD // NS
ZROWS = 128

_MESH = plsc.VectorSubcoreMesh(core_axis_name="c", subcore_axis_name="s")


def _proj_body(x_ref, w_ref, o_ref):
    o_ref[...] = jnp.dot(x_ref[...], w_ref[...],
                         preferred_element_type=jnp.float32)


def _proj(x, w):
    return pl.pallas_call(
        _proj_body,
        out_shape=jax.ShapeDtypeStruct((x.shape[0], w.shape[1]), jnp.float32),
    )(x, w)


def _relu_proj_body(p_ref, w_ref, o_ref):
    h = jnp.maximum(p_ref[0] + p_ref[1], 0.0)
    o_ref[...] = jnp.dot(h, w_ref[...], preferred_element_type=jnp.float32)


def _relu_proj(p, w):
    return pl.pallas_call(
        _relu_proj_body,
        out_shape=jax.ShapeDtypeStruct((p.shape[1], w.shape[1]), jnp.float32),
    )(p, w)


def _score_table_body(q_ref, wcat_ref, bl_ref, o_ref):
    h = jnp.maximum(q_ref[0] + q_ref[1], 0.0)
    sd = jnp.dot(h, wcat_ref[...], preferred_element_type=jnp.float32)
    col = lax.broadcasted_iota(jnp.int32, sd.shape, 1)
    o_ref[...] = sd + jnp.where(col == 0, bl_ref[0, 0], 0.0)


def _score_table(q, wcat, bl):
    return pl.pallas_call(
        _score_table_body,
        in_specs=[
            pl.BlockSpec(),
            pl.BlockSpec(),
            pl.BlockSpec(memory_space=pltpu.SMEM),
        ],
        out_shape=jax.ShapeDtypeStruct((q.shape[1], 2), jnp.float32),
    )(q, wcat, bl)


def _make_spmm(n_edges):
    assert n_edges % CHUNK == 0
    nch = n_edges // CHUNK
    nrounds = (nch + NW - 1) // NW

    @functools.partial(
        pl.kernel,
        out_type=jax.ShapeDtypeStruct((NC, NPAD, DH), jnp.float32),
        mesh=_MESH,
        compiler_params=pltpu.CompilerParams(use_tc_tiling_on_sc=False, needs_layout_passes=False),
        scratch_types=[
            pltpu.VMEM((CHUNK,), jnp.int32),
            pltpu.VMEM((CHUNK,), jnp.int32),
            pltpu.VMEM((CHUNK,), jnp.float32),
            pltpu.VMEM((CHUNK, DH), jnp.float32),
            pltpu.VMEM((ZROWS, DH), jnp.float32),
            pltpu.VMEM_SHARED((NPAD, DH), jnp.float32),
            pltpu.SemaphoreType.DMA,
        ],
    )
    def spmm(feat_hbm, col_hbm, row_hbm, val_hbm, out_hbm,
             colv, rowv, valv, gbuf, zbuf, acc, sem):
        c = lax.axis_index("c")
        s = lax.axis_index("s")
        wid = s * NC + c

        zero = jnp.zeros((L,), jnp.float32)

        def zfill(i, carry):
            for k in range(DH // L):
                zbuf[i, pl.ds(k * L, L)] = zero
            return carry

        lax.fori_loop(0, ZROWS, zfill, 0)
        for b in range(ROWS_PER_TILE // ZROWS):
            pltpu.sync_copy(zbuf, acc.at[pl.ds(s * ROWS_PER_TILE + b * ZROWS,
                                               ZROWS)])
        plsc.subcore_barrier()

        def round_body(k, carry):
            cid = k * NW + wid

            @pl.when(cid < nch)
            def _():
                base = cid * CHUNK
                pltpu.sync_copy(col_hbm.at[pl.ds(base, CHUNK)], colv)
                pltpu.sync_copy(row_hbm.at[pl.ds(base, CHUNK)], rowv)
                pltpu.sync_copy(val_hbm.at[pl.ds(base, CHUNK)], valv)
                pltpu.async_copy(feat_hbm.at[colv], gbuf, sem).wait()

                def scale(j, inner):
                    vv = valv[pl.ds(j * L, L)]
                    for i in range(L):
                        v = vv[i]
                        e = j * L + i
                        for k4 in range(DH // L):
                            sl = pl.ds(k4 * L, L)
                            gbuf[e, sl] = gbuf[e, sl] * v
                    return inner

                lax.fori_loop(0, CHUNK // L, scale, 0)
                pltpu.sync_copy(gbuf, acc.at[rowv], add=True)

            return carry

        lax.fori_loop(0, nrounds, round_body, 0)
        plsc.subcore_barrier()
        pltpu.sync_copy(acc.at[pl.ds(s * ROWS_PER_TILE, ROWS_PER_TILE)],
                        out_hbm.at[c, pl.ds(s * ROWS_PER_TILE, ROWS_PER_TILE)])

    return spmm


def _make_scorer(n_edges):
    assert n_edges % CHUNK == 0
    nch = n_edges // CHUNK
    nrounds = (nch + NW - 1) // NW

    @functools.partial(
        pl.kernel,
        out_type=jax.ShapeDtypeStruct((n_edges,), jnp.float32),
        mesh=_MESH,
        compiler_params=pltpu.CompilerParams(use_tc_tiling_on_sc=False, needs_layout_passes=False),
        scratch_types=[
            pltpu.VMEM((NPAD, 2), jnp.float32),
            pltpu.VMEM((CHUNK,), jnp.int32),
            pltpu.VMEM((CHUNK,), jnp.int32),
            pltpu.VMEM((CHUNK,), jnp.float32),
            pltpu.SemaphoreType.DMA,
        ],
    )
    def scorer(sd_hbm, src_hbm, dst_hbm, out_hbm, tbl, srcv, dstv, outv, sem):
        c = lax.axis_index("c")
        s = lax.axis_index("s")
        wid = s * NC + c
        pltpu.sync_copy(sd_hbm, tbl)
        zeros16 = jnp.zeros((L,), jnp.int32)
        ones16 = zeros16 + 1

        def round_body(k, carry):
            cid = k * NW + wid

            @pl.when(cid < nch)
            def _():
                base = cid * CHUNK
                pltpu.sync_copy(src_hbm.at[pl.ds(base, CHUNK)], srcv)
                pltpu.sync_copy(dst_hbm.at[pl.ds(base, CHUNK)], dstv)
                for j in range(CHUNK // L):
                    sl = pl.ds(j * L, L)
                    gs = plsc.load_gather(tbl, [srcv[sl], zeros16])
                    gd = plsc.load_gather(tbl, [dstv[sl], ones16])
                    t = gs + gd
                    outv[sl] = 1.0 / (1.0 + jnp.exp(-t))
                pltpu.sync_copy(outv, out_hbm.at[pl.ds(base, CHUNK)])

            return carry

        lax.fori_loop(0, nrounds, round_body, 0)

    return scorer


def kernel(x, adj_indices, adj_values, edge_index, W1, W2, Wl, bl):
    row = adj_indices[0]
    col = adj_indices[1]
    src = edge_index[0]
    dst = edge_index[1]
    n_edges = adj_values.shape[0]
    n_q = src.shape[0]

    spmm = _make_spmm(n_edges)
    scorer = _make_scorer(n_q)

    wcat = jnp.stack([Wl[0, :DH], Wl[0, DH:]], axis=1)
    bl2 = bl.reshape(1, 1)

    xw = _proj(x, W1)
    p = spmm(xw, col, row, adj_values)
    h1w = _relu_proj(p, W2)
    q = spmm(h1w, col, row, adj_values)
    sd = _score_table(q, wcat, bl2)
    return scorer(sd, src, dst)


# speedup vs baseline: 7.0589x; 1.2751x over previous
import functools

import jax
import jax.numpy as jnp
from jax import lax
from jax.experimental import pallas as pl
from jax.experimental.pallas import tpu as pltpu
from jax.experimental.pallas import tpu_sc as plsc

N = 10000
NPAD = 10240
DIN = 128
DH = 64
L = 16
NC = 2
NS = 16
NW = NC * NS
CHUNK = 128
G = 4
RE = G * CHUNK
ROWS_PER_TILE = NPAD // NS
ZROWS = 128

_MESH = plsc.VectorSubcoreMesh(core_axis_name="c", subcore_axis_name="s")
_SC_PARAMS = pltpu.CompilerParams(use_tc_tiling_on_sc=False,
                                  needs_layout_passes=False)


def _agg_proj_relu_body(p_ref, w_ref, o_ref):
    agg = jnp.concatenate([p_ref[0], p_ref[1]], axis=1)
    o_ref[...] = jnp.maximum(
        jnp.dot(agg, w_ref[...], preferred_element_type=jnp.float32), 0.0)


def _agg_proj_relu(p, w):
    return pl.pallas_call(
        _agg_proj_relu_body,
        out_shape=jax.ShapeDtypeStruct((p.shape[1], w.shape[1]), jnp.float32),
    )(p, w)


def _score_table_body(q_ref, w_ref, wcat_ref, bl_ref, o_ref):
    agg = q_ref[0] + q_ref[1]
    h = jnp.maximum(jnp.dot(agg, w_ref[...],
                            preferred_element_type=jnp.float32), 0.0)
    sd = jnp.dot(h, wcat_ref[...], preferred_element_type=jnp.float32)
    col = lax.broadcasted_iota(jnp.int32, sd.shape, 1)
    o_ref[...] = sd + jnp.where(col == 0, bl_ref[0, 0], 0.0)


def _score_table(q, w, wcat, bl):
    return pl.pallas_call(
        _score_table_body,
        in_specs=[
            pl.BlockSpec(),
            pl.BlockSpec(),
            pl.BlockSpec(),
            pl.BlockSpec(memory_space=pltpu.SMEM),
        ],
        out_shape=jax.ShapeDtypeStruct((q.shape[1], 2), jnp.float32),
    )(q, w, wcat, bl)


def _make_spmm(n_edges, df, g, split_features=False):
    re_ = g * CHUNK
    assert n_edges % re_ == 0
    nr = n_edges // re_
    workers = NS if split_features else NW
    nrt = (nr + workers - 1) // workers
    if nrt % 2:
        nrt += 1
    zrows = (8 * 1024) // df
    assert ROWS_PER_TILE % zrows == 0

    @functools.partial(
        pl.kernel,
        out_type=jax.ShapeDtypeStruct((NC, NPAD, df), jnp.float32),
        mesh=_MESH,
        compiler_params=_SC_PARAMS,
        scratch_types=[
            pltpu.VMEM((g, CHUNK), jnp.int32),
            pltpu.VMEM((g, CHUNK), jnp.int32),
            pltpu.VMEM((g, CHUNK), jnp.int32),
            pltpu.VMEM((g, CHUNK), jnp.int32),
            pltpu.VMEM((re_,), jnp.float32),
            pltpu.VMEM((re_,), jnp.float32),
            pltpu.VMEM((re_, df), jnp.float32),
            pltpu.VMEM((re_, df), jnp.float32),
            pltpu.VMEM((zrows, df), jnp.float32),
            pltpu.VMEM_SHARED((NPAD, df), jnp.float32),
            pltpu.SemaphoreType.DMA,
            pltpu.SemaphoreType.DMA,
            pltpu.SemaphoreType.DMA,
            pltpu.SemaphoreType.DMA,
            pltpu.SemaphoreType.DMA,
        ],
    )
    def spmm(feat_hbm, col_hbm, row_hbm, val_hbm, out_hbm,
             colv0, colv1, rowv0, rowv1, valv0, valv1, gbuf0, gbuf1,
             zbuf, acc, isem0, isem1, gsem0, gsem1, ssem):
        c = lax.axis_index("c")
        s = lax.axis_index("s")
        wid = s if split_features else s * NC + c
        feat_src = feat_hbm.at[c] if split_features else feat_hbm
        bufs = ((colv0, rowv0, valv0, gbuf0, isem0, gsem0),
                (colv1, rowv1, valv1, gbuf1, isem1, gsem1))

        def idx_copies(slot, r):
            colv, rowv, valv, _, isem, _ = bufs[slot]
            return (
                pltpu.make_async_copy(col_hbm.at[pl.ds(r * g, g)], colv, isem),
                pltpu.make_async_copy(row_hbm.at[pl.ds(r * g, g)], rowv, isem),
                pltpu.make_async_copy(val_hbm.at[pl.ds(r * re_, re_)], valv,
                                      isem),
            )

        def gather_copies(slot):
            colv, _, _, gbuf, _, gsem = bufs[slot]
            return tuple(
                pltpu.make_async_copy(feat_src.at[colv.at[j]],
                                      gbuf.at[pl.ds(j * CHUNK, CHUNK)], gsem)
                for j in range(g))

        def scale(slot):
            _, _, valv, gbuf, _, _ = bufs[slot]

            def body(jj, carry):
                vv = valv[pl.ds(jj * L, L)]
                for i in range(L):
                    v = vv[i]
                    e = jj * L + i
                    for k4 in range(df // L):
                        sl = pl.ds(k4 * L, L)
                        gbuf[e, sl] = gbuf[e, sl] * v
                return carry

            lax.fori_loop(0, re_ // L, body, 0)

        def scatter(slot):
            _, rowv, _, gbuf, _, _ = bufs[slot]
            hs = [pltpu.async_copy(gbuf.at[pl.ds(j * CHUNK, CHUNK)],
                                   acc.at[rowv.at[j]], ssem, add=True)
                  for j in range(g)]
            for h in hs:
                h.wait()

        r0 = wid
        r1 = wid + workers
        for h in idx_copies(0, r0):
            h.start()

        @pl.when(r1 < nr)
        def _():
            for h in idx_copies(1, r1):
                h.start()

        zero = jnp.zeros((L,), jnp.float32)

        def zfill(i, carry):
            for k in range(df // L):
                zbuf[i, pl.ds(k * L, L)] = zero
            return carry

        lax.fori_loop(0, zrows, zfill, 0)
        for b in range(ROWS_PER_TILE // zrows):
            pltpu.sync_copy(zbuf, acc.at[pl.ds(s * ROWS_PER_TILE + b * zrows,
                                               zrows)])

        for h in idx_copies(0, r0):
            h.wait()
        for h in gather_copies(0):
            h.start()
        plsc.subcore_barrier()

        def round_pair(k2, carry):
            for slot in (0, 1):
                k = 2 * k2 + slot
                r = k * workers + wid
                rn = r + workers
                rnn = r + 2 * workers

                @pl.when(rn < nr)
                def _():
                    for h in idx_copies(slot ^ 1, rn):
                        h.wait()
                    for h in gather_copies(slot ^ 1):
                        h.start()

                @pl.when(r < nr)
                def _():
                    for h in gather_copies(slot):
                        h.wait()
                    scale(slot)
                    scatter(slot)

                @pl.when(rnn < nr)
                def _():
                    for h in idx_copies(slot, rnn):
                        h.start()

            return carry

        lax.fori_loop(0, nrt // 2, round_pair, 0)
        plsc.subcore_barrier()
        pltpu.sync_copy(acc.at[pl.ds(s * ROWS_PER_TILE, ROWS_PER_TILE)],
                        out_hbm.at[c, pl.ds(s * ROWS_PER_TILE, ROWS_PER_TILE)])

    return spmm


def _make_scorer(n_edges):
    assert n_edges % RE == 0
    nr = n_edges // RE
    nrt = (nr + NW - 1) // NW
    if nrt % 2:
        nrt += 1

    @functools.partial(
        pl.kernel,
        out_type=jax.ShapeDtypeStruct((n_edges,), jnp.float32),
        mesh=_MESH,
        compiler_params=_SC_PARAMS,
        scratch_types=[
            pltpu.VMEM((NPAD, 2), jnp.float32),
            pltpu.VMEM((RE,), jnp.int32),
            pltpu.VMEM((RE,), jnp.int32),
            pltpu.VMEM((RE,), jnp.int32),
            pltpu.VMEM((RE,), jnp.int32),
            pltpu.VMEM((RE,), jnp.float32),
            pltpu.VMEM((RE,), jnp.float32),
            pltpu.SemaphoreType.DMA,
            pltpu.SemaphoreType.DMA,
            pltpu.SemaphoreType.DMA,
            pltpu.SemaphoreType.DMA,
            pltpu.SemaphoreType.DMA,
        ],
    )
    def scorer(sd_hbm, src_hbm, dst_hbm, out_hbm,
               tbl, srcv0, srcv1, dstv0, dstv1, outv0, outv1,
               isem0, isem1, osem0, osem1, tsem):
        c = lax.axis_index("c")
        s = lax.axis_index("s")
        wid = s * NC + c
        bufs = ((srcv0, dstv0, outv0, isem0, osem0),
                (srcv1, dstv1, outv1, isem1, osem1))
        zeros16 = jnp.zeros((L,), jnp.int32)
        ones16 = zeros16 + 1

        def idx_copies(slot, r):
            srcv, dstv, _, isem, _ = bufs[slot]
            return (
                pltpu.make_async_copy(src_hbm.at[pl.ds(r * RE, RE)], srcv,
                                      isem),
                pltpu.make_async_copy(dst_hbm.at[pl.ds(r * RE, RE)], dstv,
                                      isem),
            )

        def out_copy(slot, r):
            _, _, outv, _, osem = bufs[slot]
            return pltpu.make_async_copy(outv, out_hbm.at[pl.ds(r * RE, RE)],
                                         osem)

        def compute(slot):
            srcv, dstv, outv, _, _ = bufs[slot]

            def grp(jj, carry):
                sl = pl.ds(jj * L, L)
                gs = plsc.load_gather(tbl, [srcv[sl], zeros16])
                gd = plsc.load_gather(tbl, [dstv[sl], ones16])
                t = gs + gd
                outv[sl] = 1.0 / (1.0 + jnp.exp(-t))
                return carry

            lax.fori_loop(0, RE // L, grp, 0)

        r0 = wid
        r1 = wid + NW
        for h in idx_copies(0, r0):
            h.start()

        @pl.when(r1 < nr)
        def _():
            for h in idx_copies(1, r1):
                h.start()

        pltpu.async_copy(sd_hbm, tbl, tsem).wait()

        def round_pair(k2, carry):
            for slot in (0, 1):
                k = 2 * k2 + slot
                r = k * NW + wid
                rnn = r + 2 * NW

                @pl.when(r < nr)
                def _():
                    for h in idx_copies(slot, r):
                        h.wait()

                    @pl.when(r >= 2 * NW)
                    def _():
                        out_copy(slot, r - 2 * NW).wait()

                    compute(slot)
                    out_copy(slot, r).start()

                @pl.when(rnn < nr)
                def _():
                    for h in idx_copies(slot, rnn):
                        h.start()

            return carry

        lax.fori_loop(0, nrt // 2, round_pair, 0)

        for k in (nrt - 2, nrt - 1):
            r = k * NW + wid

            @pl.when(r < nr)
            def _():
                out_copy(k % 2, r).wait()

    return scorer


def kernel(x, adj_indices, adj_values, edge_index, W1, W2, Wl, bl):
    row = adj_indices[0]
    col = adj_indices[1]
    src = edge_index[0]
    dst = edge_index[1]
    n_edges = adj_values.shape[0]
    n_q = src.shape[0]

    spmm1 = _make_spmm(n_edges, DH, 4, split_features=True)
    spmm2 = _make_spmm(n_edges, DH, 4)
    scorer = _make_scorer(n_q)

    col2 = col.reshape(n_edges // CHUNK, CHUNK)
    row2 = row.reshape(n_edges // CHUNK, CHUNK)
    wcat = jnp.stack([Wl[0, :DH], Wl[0, DH:]], axis=1)
    bl2 = bl.reshape(1, 1)

    xh = x.reshape(x.shape[0], NC, DH).transpose(1, 0, 2)
    p = spmm1(xh, col2, row2, adj_values)
    h1 = _agg_proj_relu(p, W1)
    q = spmm2(h1, col2, row2, adj_values)
    sd = _score_table(q, W2, wcat, bl2)
    return scorer(sd, src, dst)


# speedup vs baseline: 14.8700x; 2.1065x over previous
import functools

import jax
import jax.numpy as jnp
from jax import lax
from jax.experimental import pallas as pl
from jax.experimental.pallas import tpu as pltpu
from jax.experimental.pallas import tpu_sc as plsc

N = 10000
NPAD = 10240
DIN = 128
DH = 64
L = 16
NC = 2
NS = 16
NW = NC * NS
CHUNK = 128
G = 4
RE = G * CHUNK
ROWS_PER_TILE = NPAD // NS
ZROWS = 128

_MESH = plsc.VectorSubcoreMesh(core_axis_name="c", subcore_axis_name="s")
_SC_PARAMS = pltpu.CompilerParams(use_tc_tiling_on_sc=False,
                                  needs_layout_passes=False)


def _agg_proj_relu_body(p_ref, w_ref, o_ref):
    agg = jnp.concatenate([p_ref[0], p_ref[1]], axis=1)
    o_ref[...] = jnp.maximum(
        jnp.dot(agg, w_ref[...], preferred_element_type=jnp.float32), 0.0)


def _agg_proj_relu(p, w):
    return pl.pallas_call(
        _agg_proj_relu_body,
        out_shape=jax.ShapeDtypeStruct((p.shape[1], w.shape[1]), jnp.float32),
    )(p, w)


def _score_table_body(q_ref, w_ref, wcat_ref, bl_ref, o_ref):
    agg = q_ref[0] + q_ref[1]
    h = jnp.maximum(jnp.dot(agg, w_ref[...],
                            preferred_element_type=jnp.float32), 0.0)
    sd = jnp.dot(h, wcat_ref[...], preferred_element_type=jnp.float32)
    col = lax.broadcasted_iota(jnp.int32, sd.shape, 1)
    o_ref[...] = sd + jnp.where(col == 0, bl_ref[0, 0], 0.0)


def _score_table(q, w, wcat, bl):
    return pl.pallas_call(
        _score_table_body,
        in_specs=[
            pl.BlockSpec(),
            pl.BlockSpec(),
            pl.BlockSpec(),
            pl.BlockSpec(memory_space=pltpu.SMEM),
        ],
        out_shape=jax.ShapeDtypeStruct((q.shape[1], 2), jnp.float32),
    )(q, w, wcat, bl)


def _make_spmm(n_edges, df, g, split_features=False):
    re_ = g * CHUNK
    assert n_edges % re_ == 0
    nr = n_edges // re_
    workers = NS if split_features else NW
    nrt = (nr + workers - 1) // workers
    if nrt % 2:
        nrt += 1
    zrows = (8 * 1024) // df
    assert ROWS_PER_TILE % zrows == 0

    @functools.partial(
        pl.kernel,
        out_type=jax.ShapeDtypeStruct((NC, NPAD, df), jnp.float32),
        mesh=_MESH,
        compiler_params=_SC_PARAMS,
        scratch_types=[
            pltpu.VMEM((g, CHUNK), jnp.int32),
            pltpu.VMEM((g, CHUNK), jnp.int32),
            pltpu.VMEM((g, CHUNK), jnp.int32),
            pltpu.VMEM((g, CHUNK), jnp.int32),
            pltpu.VMEM((re_,), jnp.float32),
            pltpu.VMEM((re_,), jnp.float32),
            pltpu.VMEM((re_, df), jnp.float32),
            pltpu.VMEM((re_, df), jnp.float32),
            pltpu.VMEM((zrows, df), jnp.float32),
            pltpu.VMEM_SHARED((NPAD, df), jnp.float32),
            pltpu.SemaphoreType.DMA,
            pltpu.SemaphoreType.DMA,
            pltpu.SemaphoreType.DMA,
            pltpu.SemaphoreType.DMA,
            pltpu.SemaphoreType.DMA,
        ],
    )
    def spmm(feat_hbm, col_hbm, row_hbm, val_hbm, out_hbm,
             colv0, colv1, rowv0, rowv1, valv0, valv1, gbuf0, gbuf1,
             zbuf, acc, isem0, isem1, gsem0, gsem1, ssem):
        c = lax.axis_index("c")
        s = lax.axis_index("s")
        wid = s if split_features else s * NC + c
        feat_src = feat_hbm.at[c] if split_features else feat_hbm
        bufs = ((colv0, rowv0, valv0, gbuf0, isem0, gsem0),
                (colv1, rowv1, valv1, gbuf1, isem1, gsem1))

        def idx_copies(slot, r):
            colv, rowv, valv, _, isem, _ = bufs[slot]
            return (
                pltpu.make_async_copy(col_hbm.at[pl.ds(r * g, g)], colv, isem),
                pltpu.make_async_copy(row_hbm.at[pl.ds(r * g, g)], rowv, isem),
                pltpu.make_async_copy(val_hbm.at[pl.ds(r * re_, re_)], valv,
                                      isem),
            )

        def gather_copies(slot):
            colv, _, _, gbuf, _, gsem = bufs[slot]
            return tuple(
                pltpu.make_async_copy(feat_src.at[colv.at[j]],
                                      gbuf.at[pl.ds(j * CHUNK, CHUNK)], gsem)
                for j in range(g))

        def scale(slot):
            _, _, valv, gbuf, _, _ = bufs[slot]

            def body(jj, carry):
                vv = valv[pl.ds(jj * L, L)]
                for i in range(L):
                    v = vv[i]
                    e = jj * L + i
                    for k4 in range(df // L):
                        sl = pl.ds(k4 * L, L)
                        gbuf[e, sl] = gbuf[e, sl] * v
                return carry

            lax.fori_loop(0, re_ // L, body, 0, unroll=2)

        def scatter(slot):
            _, rowv, _, gbuf, _, _ = bufs[slot]
            hs = [pltpu.async_copy(gbuf.at[pl.ds(j * CHUNK, CHUNK)],
                                   acc.at[rowv.at[j]], ssem, add=True)
                  for j in range(g)]
            for h in hs:
                h.wait()

        r0 = wid
        r1 = wid + workers
        for h in idx_copies(0, r0):
            h.start()

        @pl.when(r1 < nr)
        def _():
            for h in idx_copies(1, r1):
                h.start()

        zero = jnp.zeros((L,), jnp.float32)

        def zfill(i, carry):
            for k in range(df // L):
                zbuf[i, pl.ds(k * L, L)] = zero
            return carry

        lax.fori_loop(0, zrows, zfill, 0)
        for b in range(ROWS_PER_TILE // zrows):
            pltpu.sync_copy(zbuf, acc.at[pl.ds(s * ROWS_PER_TILE + b * zrows,
                                               zrows)])

        for h in idx_copies(0, r0):
            h.wait()
        for h in gather_copies(0):
            h.start()
        plsc.subcore_barrier()

        def round_pair(k2, carry):
            for slot in (0, 1):
                k = 2 * k2 + slot
                r = k * workers + wid
                rn = r + workers
                rnn = r + 2 * workers

                @pl.when(rn < nr)
                def _():
                    for h in idx_copies(slot ^ 1, rn):
                        h.wait()
                    for h in gather_copies(slot ^ 1):
                        h.start()

                @pl.when(r < nr)
                def _():
                    for h in gather_copies(slot):
                        h.wait()
                    scale(slot)
                    scatter(slot)

                @pl.when(rnn < nr)
                def _():
                    for h in idx_copies(slot, rnn):
                        h.start()

            return carry

        lax.fori_loop(0, nrt // 2, round_pair, 0)
        plsc.subcore_barrier()
        pltpu.sync_copy(acc.at[pl.ds(s * ROWS_PER_TILE, ROWS_PER_TILE)],
                        out_hbm.at[c, pl.ds(s * ROWS_PER_TILE, ROWS_PER_TILE)])

    return spmm


def _make_scorer(n_edges):
    assert n_edges % RE == 0
    nr = n_edges // RE
    nrt = (nr + NW - 1) // NW
    if nrt % 2:
        nrt += 1

    @functools.partial(
        pl.kernel,
        out_type=jax.ShapeDtypeStruct((n_edges,), jnp.float32),
        mesh=_MESH,
        compiler_params=_SC_PARAMS,
        scratch_types=[
            pltpu.VMEM((NPAD, 2), jnp.float32),
            pltpu.VMEM((RE,), jnp.int32),
            pltpu.VMEM((RE,), jnp.int32),
            pltpu.VMEM((RE,), jnp.int32),
            pltpu.VMEM((RE,), jnp.int32),
            pltpu.VMEM((RE,), jnp.float32),
            pltpu.VMEM((RE,), jnp.float32),
            pltpu.SemaphoreType.DMA,
            pltpu.SemaphoreType.DMA,
            pltpu.SemaphoreType.DMA,
            pltpu.SemaphoreType.DMA,
            pltpu.SemaphoreType.DMA,
        ],
    )
    def scorer(sd_hbm, src_hbm, dst_hbm, out_hbm,
               tbl, srcv0, srcv1, dstv0, dstv1, outv0, outv1,
               isem0, isem1, osem0, osem1, tsem):
        c = lax.axis_index("c")
        s = lax.axis_index("s")
        wid = s * NC + c
        bufs = ((srcv0, dstv0, outv0, isem0, osem0),
                (srcv1, dstv1, outv1, isem1, osem1))
        zeros16 = jnp.zeros((L,), jnp.int32)
        ones16 = zeros16 + 1

        def idx_copies(slot, r):
            srcv, dstv, _, isem, _ = bufs[slot]
            return (
                pltpu.make_async_copy(src_hbm.at[pl.ds(r * RE, RE)], srcv,
                                      isem),
                pltpu.make_async_copy(dst_hbm.at[pl.ds(r * RE, RE)], dstv,
                                      isem),
            )

        def out_copy(slot, r):
            _, _, outv, _, osem = bufs[slot]
            return pltpu.make_async_copy(outv, out_hbm.at[pl.ds(r * RE, RE)],
                                         osem)

        def compute(slot):
            srcv, dstv, outv, _, _ = bufs[slot]

            def grp(jj, carry):
                sl = pl.ds(jj * L, L)
                gs = plsc.load_gather(tbl, [srcv[sl], zeros16])
                gd = plsc.load_gather(tbl, [dstv[sl], ones16])
                t = gs + gd
                outv[sl] = 1.0 / (1.0 + jnp.exp(-t))
                return carry

            lax.fori_loop(0, RE // L, grp, 0)

        r0 = wid
        r1 = wid + NW
        for h in idx_copies(0, r0):
            h.start()

        @pl.when(r1 < nr)
        def _():
            for h in idx_copies(1, r1):
                h.start()

        pltpu.async_copy(sd_hbm, tbl, tsem).wait()

        def round_pair(k2, carry):
            for slot in (0, 1):
                k = 2 * k2 + slot
                r = k * NW + wid
                rnn = r + 2 * NW

                @pl.when(r < nr)
                def _():
                    for h in idx_copies(slot, r):
                        h.wait()

                    @pl.when(r >= 2 * NW)
                    def _():
                        out_copy(slot, r - 2 * NW).wait()

                    compute(slot)
                    out_copy(slot, r).start()

                @pl.when(rnn < nr)
                def _():
                    for h in idx_copies(slot, rnn):
                        h.start()

            return carry

        lax.fori_loop(0, nrt // 2, round_pair, 0)

        for k in (nrt - 2, nrt - 1):
            r = k * NW + wid

            @pl.when(r < nr)
            def _():
                out_copy(k % 2, r).wait()

    return scorer


def kernel(x, adj_indices, adj_values, edge_index, W1, W2, Wl, bl):
    row = adj_indices[0]
    col = adj_indices[1]
    src = edge_index[0]
    dst = edge_index[1]
    n_edges = adj_values.shape[0]
    n_q = src.shape[0]

    spmm1 = _make_spmm(n_edges, DH, 4, split_features=True)
    spmm2 = _make_spmm(n_edges, DH, 4)
    scorer = _make_scorer(n_q)

    col2 = col.reshape(n_edges // CHUNK, CHUNK)
    row2 = row.reshape(n_edges // CHUNK, CHUNK)
    wcat = jnp.stack([Wl[0, :DH], Wl[0, DH:]], axis=1)
    bl2 = bl.reshape(1, 1)

    xh = x.reshape(x.shape[0], NC, DH).transpose(1, 0, 2)
    p = spmm1(xh, col2, row2, adj_values)
    h1 = _agg_proj_relu(p, W1)
    q = spmm2(h1, col2, row2, adj_values)
    sd = _score_table(q, W2, wcat, bl2)
    return scorer(sd, src, dst)
